# baseline (speedup 1.0000x reference)
import functools

import jax
import jax.numpy as jnp
from jax import lax
from jax.experimental import pallas as pl
from jax.experimental.pallas import tpu as pltpu
from jax.experimental.pallas import tpu_sc as plsc

_NUM_LABELS = 1000
_EMBED = 64
_NC = 2
_NS = 16
_NW = _NC * _NS
_CHUNK = 128


def _make_gather(B, S):
    n_idx = B * S
    assert n_idx % (_NW * _CHUNK) == 0 and B % _NW == 0
    per_w = n_idx // _NW
    n_chunks = per_w // _CHUNK
    b_per_w = B // _NW
    mesh = plsc.VectorSubcoreMesh(
        core_axis_name="c", subcore_axis_name="s",
        num_cores=_NC, num_subcores=_NS,
    )

    @functools.partial(
        pl.kernel,
        out_type=jax.ShapeDtypeStruct((B, S, _EMBED), jnp.float32),
        mesh=mesh,
        scratch_types=[
            pltpu.VMEM((per_w,), jnp.int32),
            pltpu.VMEM((per_w, _EMBED), jnp.float32),
            pltpu.SemaphoreType.DMA,
            pltpu.SemaphoreType.DMA,
        ],
        compiler_params=pltpu.CompilerParams(use_tc_tiling_on_sc=False),
    )
    def gather(table_hbm, idx_hbm, out_hbm, idx_v, rows_v, gsem, wsem):
        wid = lax.axis_index("s") * _NC + lax.axis_index("c")
        pltpu.sync_copy(idx_hbm.at[pl.ds(wid * per_w, per_w)], idx_v)
        pltpu.async_copy(table_hbm.at[idx_v], rows_v, gsem).wait()
        writes = [
            pltpu.async_copy(
                rows_v.at[pl.ds(p * S, S)],
                out_hbm.at[wid * b_per_w + p],
                wsem,
            )
            for p in range(b_per_w)
        ]
        for cp in writes:
            cp.wait()

    return gather


def kernel(X, W, b):
    B, S, _ = X.shape
    table = W.T
    idx = X.reshape(B * S).astype(jnp.int32)
    rows = _make_gather(B, S)(table, idx)
    return rows + b[None, None, :]


# speedup vs baseline: 1.1037x; 1.1037x over previous
import functools

import jax
import jax.numpy as jnp
from jax import lax
from jax.experimental import pallas as pl
from jax.experimental.pallas import tpu as pltpu
from jax.experimental.pallas import tpu_sc as plsc

_NUM_LABELS = 1000
_EMBED = 64
_NC = 2
_NS = 16
_NW = _NC * _NS
_CHUNK = 128


def _make_gather(B, S):
    n_idx = B * S
    assert n_idx % _NW == 0 and B % _NW == 0
    per_w = n_idx // _NW
    assert per_w % 8 == 0
    b_per_w = B // _NW
    mesh = plsc.VectorSubcoreMesh(
        core_axis_name="c", subcore_axis_name="s",
        num_cores=_NC, num_subcores=_NS,
    )

    @functools.partial(
        pl.kernel,
        out_type=jax.ShapeDtypeStruct((B, S, _EMBED), jnp.float32),
        mesh=mesh,
        scratch_types=[
            pltpu.VMEM((per_w,), jnp.int32),
            pltpu.VMEM((per_w, _EMBED), jnp.float32),
            pltpu.SemaphoreType.DMA,
            pltpu.SemaphoreType.DMA,
        ],
        compiler_params=pltpu.CompilerParams(use_tc_tiling_on_sc=False),
    )
    def gather(table_hbm, idx_hbm, out_hbm, idx_v, rows_v, gsem, wsem):
        wid = lax.axis_index("s") * _NC + lax.axis_index("c")
        pltpu.sync_copy(idx_hbm.at[pl.ds(wid * per_w, per_w)], idx_v)
        pltpu.async_copy(table_hbm.at[idx_v], rows_v, gsem).wait()
        writes = [
            pltpu.async_copy(
                rows_v.at[pl.ds(p * S, S)],
                out_hbm.at[wid * b_per_w + p],
                wsem,
            )
            for p in range(b_per_w)
        ]
        for cp in writes:
            cp.wait()

    return gather


def kernel(X, W, b):
    B, S, _ = X.shape
    table = W.T + b[None, :]
    idx = X.reshape(B * S).astype(jnp.int32)
    half = B // 2
    g = _make_gather(half, S)
    outA = g(table, idx[: half * S])
    outB = g(table, idx[half * S:])
    return jnp.concatenate([outA, outB], axis=0)


# speedup vs baseline: 1.1489x; 1.0409x over previous
import functools

import jax
import jax.numpy as jnp
from jax import lax
from jax.experimental import pallas as pl
from jax.experimental.pallas import tpu as pltpu
from jax.experimental.pallas import tpu_sc as plsc

_NUM_LABELS = 1000
_EMBED = 64
_NC = 2
_NS = 16
_NW = _NC * _NS
_CHUNK = 128


def _make_gather(B, S):
    n_idx = B * S
    assert n_idx % _NW == 0 and B % _NW == 0
    per_w = n_idx // _NW
    assert per_w % 8 == 0
    b_per_w = B // _NW
    mesh = plsc.VectorSubcoreMesh(
        core_axis_name="c", subcore_axis_name="s",
        num_cores=_NC, num_subcores=_NS,
    )

    @functools.partial(
        pl.kernel,
        out_type=jax.ShapeDtypeStruct((B, S, _EMBED), jnp.float32),
        mesh=mesh,
        scratch_types=[
            pltpu.VMEM((per_w,), jnp.int32),
            pltpu.VMEM((per_w, _EMBED), jnp.float32),
            pltpu.SemaphoreType.DMA,
            pltpu.SemaphoreType.DMA,
        ],
        compiler_params=pltpu.CompilerParams(use_tc_tiling_on_sc=False),
    )
    def gather(table_hbm, idx_hbm, out_hbm, idx_v, rows_v, gsem, wsem):
        wid = lax.axis_index("s") * _NC + lax.axis_index("c")
        pltpu.sync_copy(idx_hbm.at[pl.ds(wid * per_w, per_w)], idx_v)
        pltpu.async_copy(table_hbm.at[idx_v], rows_v, gsem).wait()
        writes = [
            pltpu.async_copy(
                rows_v.at[pl.ds(p * S, S)],
                out_hbm.at[wid * b_per_w + p],
                wsem,
            )
            for p in range(b_per_w)
        ]
        for cp in writes:
            cp.wait()

    return gather


def kernel(X, W, b):
    B, S, _ = X.shape
    table = W.T + b[None, :]
    idx = X.reshape(B * S).astype(jnp.int32)
    return _make_gather(B, S)(table, idx)
